# baseline (speedup 1.0000x reference)
import functools

import jax
import jax.numpy as jnp
from jax import lax
from jax.experimental import pallas as pl
from jax.experimental.pallas import tpu as pltpu
from jax.experimental.pallas import tpu_sc as plsc

D_VOCAB = 1_000_000
D_MODEL = 64
N_TOK = 4096 * 50

NC, NS = 2, 16
NW = NC * NS
TOK_PER_W = N_TOK // NW
CHUNK = 128
NCHUNK = TOK_PER_W // CHUNK

TP_BLK = 8192


def _tp_body(w_ref, o_ref):
    o_ref[...] = w_ref[...].T


def _transpose(w_e):
    grid = pl.cdiv(D_VOCAB, TP_BLK)
    return pl.pallas_call(
        _tp_body,
        grid=(grid,),
        in_specs=[pl.BlockSpec((D_MODEL, TP_BLK), lambda i: (0, i))],
        out_specs=pl.BlockSpec((TP_BLK, D_MODEL), lambda i: (i, 0)),
        out_shape=jax.ShapeDtypeStruct((D_VOCAB, D_MODEL), jnp.float32),
    )(w_e)


@functools.partial(
    pl.kernel,
    out_type=jax.ShapeDtypeStruct((N_TOK, D_MODEL), jnp.float32),
    mesh=plsc.VectorSubcoreMesh(core_axis_name="c", subcore_axis_name="s"),
    scratch_types=[
        pltpu.VMEM((NCHUNK, CHUNK), jnp.int32),
        pltpu.VMEM((CHUNK, D_MODEL), jnp.float32),
        pltpu.SemaphoreType.DMA,
    ],
    compiler_params=pltpu.CompilerParams(use_tc_tiling_on_sc=False),
)
def _sc_gather(x_hbm, wt_hbm, out_hbm, idx_v, rows_v, sem):
    w = lax.axis_index("s") * NC + lax.axis_index("c")
    pltpu.sync_copy(x_hbm.at[w], idx_v)

    def body(c, carry):
        pltpu.async_copy(wt_hbm.at[idx_v.at[c]], rows_v, sem).wait()
        pltpu.sync_copy(rows_v, out_hbm.at[pl.ds(w * TOK_PER_W + c * CHUNK, CHUNK)])
        return carry

    lax.fori_loop(0, NCHUNK, body, 0)


def kernel(x, W_E):
    w_t = _transpose(W_E)
    x3 = x.reshape(NW, NCHUNK, CHUNK).astype(jnp.int32)
    out = _sc_gather(x3, w_t)
    return out.reshape(4096, 50, D_MODEL)
